# baseline (speedup 1.0000x reference)
import functools

import jax
import jax.numpy as jnp
from jax import lax
from jax.experimental import pallas as pl
from jax.experimental.pallas import tpu as pltpu
from jax.experimental.pallas import tpu_sc as plsc

N = 10000
D = 128
DP = 144
E = 320000
NC, NS = 2, 16
NW = NC * NS
EPW = E // NW
CHUNK = 80
NCHUNK = EPW // CHUNK
ZP = 10240
RPS = ZP // NS


def _make_agg(d, chunk, nbuf):
    mesh = plsc.VectorSubcoreMesh(core_axis_name="c", subcore_axis_name="s")
    nchunk = EPW // chunk

    @functools.partial(
        pl.kernel,
        out_type=jax.ShapeDtypeStruct((NC, ZP, d), jnp.float32),
        mesh=mesh,
        compiler_params=pltpu.CompilerParams(use_tc_tiling_on_sc=False),
        scratch_types=[
            pltpu.VMEM((nchunk, chunk), jnp.int32),
            pltpu.VMEM((nchunk, chunk), jnp.int32),
            [pltpu.VMEM((chunk, d), jnp.float32)] * nbuf,
            pltpu.VMEM_SHARED((ZP, d), jnp.float32),
            [pltpu.SemaphoreType.DMA] * nbuf,
        ],
    )
    def agg(table_hbm, src_hbm, dst_hbm, out_hbm, sidx_v, didx_v, rows,
            acc_sh, gsem):
        rows_a = rows[0]
        cid = lax.axis_index("c")
        sid = lax.axis_index("s")
        wid = sid * NC + cid

        @pl.loop(0, chunk)
        def _(r):
            @pl.loop(0, d, step=16)
            def _(j):
                rows_a[r, pl.ds(j, 16)] = jnp.zeros((16,), jnp.float32)

        base = pl.multiple_of(sid * RPS, 8)

        @pl.loop(0, RPS // chunk)
        def _(k):
            pltpu.sync_copy(
                rows_a, acc_sh.at[pl.ds(pl.multiple_of(base + k * chunk, 8), chunk)])

        rem = RPS % chunk
        if rem:
            pltpu.sync_copy(
                rows_a.at[pl.ds(0, rem)],
                acc_sh.at[pl.ds(base + (RPS // chunk) * chunk, rem)])

        plsc.subcore_barrier()

        pltpu.sync_copy(src_hbm.at[wid], sidx_v)
        pltpu.sync_copy(dst_hbm.at[wid], didx_v)

        @pl.loop(0, nchunk // nbuf)
        def _(p):
            i0 = nbuf * p
            gs = [pltpu.async_copy(table_hbm.at[sidx_v.at[i0 + k]], rows[k],
                                   gsem[k]) for k in range(nbuf)]
            for k in range(nbuf):
                gs[k].wait()
                pltpu.sync_copy(rows[k], acc_sh.at[didx_v.at[i0 + k]], add=True)

        for i in range((nchunk // nbuf) * nbuf, nchunk):
            pltpu.sync_copy(table_hbm.at[sidx_v.at[i]], rows_a)
            pltpu.sync_copy(rows_a, acc_sh.at[didx_v.at[i]], add=True)

        plsc.subcore_barrier()
        pltpu.sync_copy(acc_sh.at[pl.ds(base, RPS)],
                        out_hbm.at[cid].at[pl.ds(base, RPS)])

    return agg


C1, C2 = 40, 80
_agg_l1 = _make_agg(DP, C1, 3)
_agg_l2 = _make_agg(D, C1, 4)


def _dense_body(transition):
    def body(cnt_ref, part_ref, xin_ref, wl_ref, bl_ref, wr_ref, gam_ref,
             bet_ref, out_ref):
        s = part_ref[0, :, :D] + part_ref[1, :, :D]
        cnt = cnt_ref[0, :, D:D + 1] + cnt_ref[1, :, D:D + 1]
        mean = s / jnp.maximum(cnt, 1.0)
        dn = (((1,), (1,)), ((), ()))
        out = (lax.dot_general(mean, wl_ref[...], dn,
                               preferred_element_type=jnp.float32)
               + bl_ref[...]
               + lax.dot_general(xin_ref[...], wr_ref[...], dn,
                                 preferred_element_type=jnp.float32))
        nrm = jnp.sqrt(jnp.sum(out * out, axis=1, keepdims=True))
        out = out / jnp.maximum(nrm, 1e-12)
        if transition:
            scale = gam_ref[...] * (1.0 / jnp.sqrt(jnp.float32(1.0 + 1e-5)))
            out = jnp.maximum(out, 0.0) * scale + bet_ref[...]
        out_ref[...] = out

    return body


BLK = 1000


def _dense(cnt_part, part, xin, wl, bl, wr, gam, bet, transition):
    part_d = part.shape[-1]
    grid = (N // BLK,)
    return pl.pallas_call(
        _dense_body(transition),
        grid=grid,
        in_specs=[
            pl.BlockSpec((NC, BLK, DP), lambda i: (0, i, 0)),
            pl.BlockSpec((NC, BLK, part_d), lambda i: (0, i, 0)),
            pl.BlockSpec((BLK, D), lambda i: (i, 0)),
            pl.BlockSpec((D, D), lambda i: (0, 0)),
            pl.BlockSpec((1, D), lambda i: (0, 0)),
            pl.BlockSpec((D, D), lambda i: (0, 0)),
            pl.BlockSpec((1, D), lambda i: (0, 0)),
            pl.BlockSpec((1, D), lambda i: (0, 0)),
        ],
        out_specs=pl.BlockSpec((BLK, D), lambda i: (i, 0)),
        out_shape=jax.ShapeDtypeStruct((N, D), jnp.float32),
    )(cnt_part, part, xin, wl, bl, wr, gam, bet)


def kernel(x, edge_index, W1l, b1l, W1r, bn_gamma, bn_beta, W2l, b2l, W2r):
    src = edge_index[0].astype(jnp.int32)
    dst = edge_index[1].astype(jnp.int32)
    src1 = src.reshape(NW, EPW // C1, C1)
    dst1 = dst.reshape(NW, EPW // C1, C1)
    src2 = src.reshape(NW, EPW // C2, C2)
    dst2 = dst.reshape(NW, EPW // C2, C2)

    pad = jnp.zeros((N, DP - D - 1), jnp.float32)
    ones = jnp.ones((N, 1), jnp.float32)
    xaug = jnp.concatenate([x, ones, pad], axis=1)

    part1 = _agg_l1(xaug, src1, dst1)
    h = _dense(part1, part1, x, W1l, b1l.reshape(1, D), W1r,
               bn_gamma.reshape(1, D), bn_beta.reshape(1, D), True)
    part2 = _agg_l2(h, src1, dst1)
    out = _dense(part1, part2, h, W2l, b2l.reshape(1, D), W2r,
                 bn_gamma.reshape(1, D), bn_beta.reshape(1, D), False)
    return out


# speedup vs baseline: 1.0578x; 1.0578x over previous
import functools

import jax
import jax.numpy as jnp
from jax import lax
from jax.experimental import pallas as pl
from jax.experimental.pallas import tpu as pltpu
from jax.experimental.pallas import tpu_sc as plsc

N = 10000
D = 128
DP = 144
E = 320000
NC, NS = 2, 16
NW = NC * NS
EPW = E // NW
CHUNK = 80
NCHUNK = EPW // CHUNK
ZP = 10240
RPS = ZP // NS


def _make_agg(d, chunk, nbuf, split):
    mesh = plsc.VectorSubcoreMesh(core_axis_name="c", subcore_axis_name="s")
    nchunk = EPW // chunk
    if split:
        out_type = (jax.ShapeDtypeStruct((NC, ZP, D), jnp.float32),
                    jax.ShapeDtypeStruct((NC, ZP, d - D), jnp.float32))
    else:
        out_type = jax.ShapeDtypeStruct((NC, ZP, d), jnp.float32)

    @functools.partial(
        pl.kernel,
        out_type=out_type,
        mesh=mesh,
        compiler_params=pltpu.CompilerParams(use_tc_tiling_on_sc=False),
        scratch_types=[
            pltpu.VMEM((nchunk, chunk), jnp.int32),
            pltpu.VMEM((nchunk, chunk), jnp.int32),
            [pltpu.VMEM((chunk, d), jnp.float32)] * nbuf,
            pltpu.VMEM_SHARED((ZP, d), jnp.float32),
            [pltpu.SemaphoreType.DMA] * nbuf,
        ],
    )
    def agg(table_hbm, src_hbm, dst_hbm, *rest):
        if split:
            outf_hbm, outc_hbm, sidx_v, didx_v, rows, acc_sh, gsem = rest
        else:
            outf_hbm, sidx_v, didx_v, rows, acc_sh, gsem = rest
        rows_a = rows[0]
        cid = lax.axis_index("c")
        sid = lax.axis_index("s")
        wid = sid * NC + cid

        @pl.loop(0, chunk)
        def _(r):
            @pl.loop(0, d, step=16)
            def _(j):
                rows_a[r, pl.ds(j, 16)] = jnp.zeros((16,), jnp.float32)

        base = pl.multiple_of(sid * RPS, 8)

        @pl.loop(0, RPS // chunk)
        def _(k):
            pltpu.sync_copy(
                rows_a, acc_sh.at[pl.ds(pl.multiple_of(base + k * chunk, 8), chunk)])

        rem = RPS % chunk
        if rem:
            pltpu.sync_copy(
                rows_a.at[pl.ds(0, rem)],
                acc_sh.at[pl.ds(base + (RPS // chunk) * chunk, rem)])

        plsc.subcore_barrier()

        pltpu.sync_copy(src_hbm.at[wid], sidx_v)
        pltpu.sync_copy(dst_hbm.at[wid], didx_v)

        @pl.loop(0, nchunk // nbuf)
        def _(p):
            i0 = nbuf * p
            gs = [pltpu.async_copy(table_hbm.at[sidx_v.at[i0 + k]], rows[k],
                                   gsem[k]) for k in range(nbuf)]
            for k in range(nbuf):
                gs[k].wait()
                pltpu.sync_copy(rows[k], acc_sh.at[didx_v.at[i0 + k]], add=True)

        for i in range((nchunk // nbuf) * nbuf, nchunk):
            pltpu.sync_copy(table_hbm.at[sidx_v.at[i]], rows_a)
            pltpu.sync_copy(rows_a, acc_sh.at[didx_v.at[i]], add=True)

        plsc.subcore_barrier()
        if split:
            pltpu.sync_copy(acc_sh.at[pl.ds(base, RPS), pl.ds(0, D)],
                            outf_hbm.at[cid].at[pl.ds(base, RPS)])
            pltpu.sync_copy(acc_sh.at[pl.ds(base, RPS), pl.ds(D, d - D)],
                            outc_hbm.at[cid].at[pl.ds(base, RPS)])
        else:
            pltpu.sync_copy(acc_sh.at[pl.ds(base, RPS)],
                            outf_hbm.at[cid].at[pl.ds(base, RPS)])

    return agg


C1, C2 = 40, 80
_agg_l1 = _make_agg(DP, C1, 3, True)
_agg_l2 = _make_agg(D, C2, 2, False)


def _dense_body(transition):
    def body(cnt_ref, part_ref, xin_ref, wl_ref, bl_ref, wr_ref, gam_ref,
             bet_ref, out_ref):
        s = part_ref[0] + part_ref[1]
        cnt = cnt_ref[0, :, 0:1] + cnt_ref[1, :, 0:1]
        mean = s / jnp.maximum(cnt, 1.0)
        dn = (((1,), (1,)), ((), ()))
        out = (lax.dot_general(mean, wl_ref[...], dn,
                               preferred_element_type=jnp.float32)
               + bl_ref[...]
               + lax.dot_general(xin_ref[...], wr_ref[...], dn,
                                 preferred_element_type=jnp.float32))
        nrm = jnp.sqrt(jnp.sum(out * out, axis=1, keepdims=True))
        out = out / jnp.maximum(nrm, 1e-12)
        if transition:
            scale = gam_ref[...] * (1.0 / jnp.sqrt(jnp.float32(1.0 + 1e-5)))
            out = jnp.maximum(out, 0.0) * scale + bet_ref[...]
        out_ref[...] = out

    return body


BLK = 1000
CW = DP - D


def _dense(cnt_part, part, xin, wl, bl, wr, gam, bet, transition):
    grid = (N // BLK,)
    return pl.pallas_call(
        _dense_body(transition),
        grid=grid,
        in_specs=[
            pl.BlockSpec((NC, BLK, CW), lambda i: (0, i, 0)),
            pl.BlockSpec((NC, BLK, D), lambda i: (0, i, 0)),
            pl.BlockSpec((BLK, D), lambda i: (i, 0)),
            pl.BlockSpec((D, D), lambda i: (0, 0)),
            pl.BlockSpec((1, D), lambda i: (0, 0)),
            pl.BlockSpec((D, D), lambda i: (0, 0)),
            pl.BlockSpec((1, D), lambda i: (0, 0)),
            pl.BlockSpec((1, D), lambda i: (0, 0)),
        ],
        out_specs=pl.BlockSpec((BLK, D), lambda i: (i, 0)),
        out_shape=jax.ShapeDtypeStruct((N, D), jnp.float32),
    )(cnt_part, part, xin, wl, bl, wr, gam, bet)


def kernel(x, edge_index, W1l, b1l, W1r, bn_gamma, bn_beta, W2l, b2l, W2r):
    src = edge_index[0].astype(jnp.int32)
    dst = edge_index[1].astype(jnp.int32)
    src1 = src.reshape(NW, EPW // C1, C1)
    dst1 = dst.reshape(NW, EPW // C1, C1)
    src2 = src.reshape(NW, EPW // C2, C2)
    dst2 = dst.reshape(NW, EPW // C2, C2)

    pad = jnp.zeros((N, DP - D - 1), jnp.float32)
    ones = jnp.ones((N, 1), jnp.float32)
    xaug = jnp.concatenate([x, ones, pad], axis=1)

    part1, cntp = _agg_l1(xaug, src1, dst1)
    h = _dense(cntp, part1, x, W1l, b1l.reshape(1, D), W1r,
               bn_gamma.reshape(1, D), bn_beta.reshape(1, D), True)
    part2 = _agg_l2(h, src2, dst2)
    out = _dense(cntp, part2, h, W2l, b2l.reshape(1, D), W2r,
                 bn_gamma.reshape(1, D), bn_beta.reshape(1, D), False)
    return out


# speedup vs baseline: 1.0690x; 1.0106x over previous
import functools

import jax
import jax.numpy as jnp
from jax import lax
from jax.experimental import pallas as pl
from jax.experimental.pallas import tpu as pltpu
from jax.experimental.pallas import tpu_sc as plsc

N = 10000
D = 128
DP = 144
E = 320000
NC, NS = 2, 16
NW = NC * NS
EPW = E // NW
CHUNK = 80
NCHUNK = EPW // CHUNK
CW = 16
ZP = 10240
RPS = ZP // NS


def _make_agg(chunk, nbuf, count):
    mesh = plsc.VectorSubcoreMesh(core_axis_name="c", subcore_axis_name="s")
    nchunk = EPW // chunk
    out_type = jax.ShapeDtypeStruct((NC, ZP, D), jnp.float32)
    scratch = [
        pltpu.VMEM((EPW,), jnp.int32),
        pltpu.VMEM((EPW,), jnp.int32),
        [pltpu.VMEM((chunk, D), jnp.float32)] * nbuf,
        pltpu.VMEM_SHARED((ZP, D), jnp.float32),
        [pltpu.SemaphoreType.DMA] * nbuf,
    ]
    if count:
        out_type = (out_type, jax.ShapeDtypeStruct((NC, ZP, CW), jnp.float32))
        scratch += [
            pltpu.VMEM((chunk, CW), jnp.float32),
            pltpu.VMEM_SHARED((ZP, CW), jnp.float32),
        ]

    @functools.partial(pl.kernel, out_type=out_type, mesh=mesh,
                       compiler_params=pltpu.CompilerParams(
                           use_tc_tiling_on_sc=False),
                       scratch_types=scratch)
    def agg(table_hbm, src_hbm, dst_hbm, *rest):
        if count:
            (outf_hbm, outc_hbm, sidx_v, didx_v, rows, acc_sh, gsem,
             ones_v, cnt_sh) = rest
        else:
            outf_hbm, sidx_v, didx_v, rows, acc_sh, gsem = rest
        rows_a = rows[0]
        cid = lax.axis_index("c")
        sid = lax.axis_index("s")
        wid = sid * NC + cid

        @pl.loop(0, chunk)
        def _(r):
            @pl.loop(0, D, step=16)
            def _(j):
                rows_a[r, pl.ds(j, 16)] = jnp.zeros((16,), jnp.float32)

        base = pl.multiple_of(sid * RPS, 8)

        @pl.loop(0, RPS // chunk)
        def _(k):
            pltpu.sync_copy(
                rows_a, acc_sh.at[pl.ds(pl.multiple_of(base + k * chunk, 8), chunk)])

        if count:
            e0 = jnp.where(lax.iota(jnp.int32, 16) == 0,
                           jnp.float32(1.0), jnp.float32(0.0))

            @pl.loop(0, chunk)
            def _(r):
                ones_v[r, pl.ds(0, 16)] = e0

            @pl.loop(0, RPS // chunk)
            def _(k):
                pltpu.sync_copy(
                    rows_a.at[pl.ds(0, chunk), pl.ds(0, CW)],
                    cnt_sh.at[pl.ds(pl.multiple_of(base + k * chunk, 8), chunk)])

        plsc.subcore_barrier()

        pltpu.sync_copy(src_hbm.at[pl.ds(wid * EPW, EPW)], sidx_v)
        pltpu.sync_copy(dst_hbm.at[pl.ds(wid * EPW, EPW)], didx_v)

        @pl.loop(0, nchunk // nbuf)
        def _(p):
            i0 = nbuf * p
            gs = [pltpu.async_copy(
                      table_hbm.at[sidx_v.at[pl.ds((i0 + k) * chunk, chunk)]],
                      rows[k], gsem[k]) for k in range(nbuf)]
            for k in range(nbuf):
                gs[k].wait()
                didx = didx_v.at[pl.ds((i0 + k) * chunk, chunk)]
                pltpu.sync_copy(rows[k], acc_sh.at[didx], add=True)
                if count:
                    pltpu.sync_copy(ones_v, cnt_sh.at[didx], add=True)

        for i in range((nchunk // nbuf) * nbuf, nchunk):
            didx = didx_v.at[pl.ds(i * chunk, chunk)]
            pltpu.sync_copy(table_hbm.at[sidx_v.at[pl.ds(i * chunk, chunk)]],
                            rows_a)
            pltpu.sync_copy(rows_a, acc_sh.at[didx], add=True)
            if count:
                pltpu.sync_copy(ones_v, cnt_sh.at[didx], add=True)

        plsc.subcore_barrier()
        pltpu.sync_copy(acc_sh.at[pl.ds(base, RPS)],
                        outf_hbm.at[cid].at[pl.ds(base, RPS)])
        if count:
            pltpu.sync_copy(cnt_sh.at[pl.ds(base, RPS)],
                            outc_hbm.at[cid].at[pl.ds(base, RPS)])

    return agg


C1, C2 = 40, 80
_agg_l1 = _make_agg(C1, 3, True)
_agg_l2 = _make_agg(C2, 2, False)


def _dense_body(transition):
    def body(cnt_ref, part_ref, xin_ref, wl_ref, bl_ref, wr_ref, gam_ref,
             bet_ref, out_ref):
        s = part_ref[0] + part_ref[1]
        cnt = cnt_ref[0, :, 0:1] + cnt_ref[1, :, 0:1]
        mean = s / jnp.maximum(cnt, 1.0)
        dn = (((1,), (1,)), ((), ()))
        out = (lax.dot_general(mean, wl_ref[...], dn,
                               preferred_element_type=jnp.float32)
               + bl_ref[...]
               + lax.dot_general(xin_ref[...], wr_ref[...], dn,
                                 preferred_element_type=jnp.float32))
        nrm = jnp.sqrt(jnp.sum(out * out, axis=1, keepdims=True))
        out = out / jnp.maximum(nrm, 1e-12)
        if transition:
            scale = gam_ref[...] * (1.0 / jnp.sqrt(jnp.float32(1.0 + 1e-5)))
            out = jnp.maximum(out, 0.0) * scale + bet_ref[...]
        out_ref[...] = out

    return body


BLK = 1000


def _dense(cnt_part, part, xin, wl, bl, wr, gam, bet, transition):
    grid = (N // BLK,)
    return pl.pallas_call(
        _dense_body(transition),
        grid=grid,
        in_specs=[
            pl.BlockSpec((NC, BLK, CW), lambda i: (0, i, 0)),
            pl.BlockSpec((NC, BLK, D), lambda i: (0, i, 0)),
            pl.BlockSpec((BLK, D), lambda i: (i, 0)),
            pl.BlockSpec((D, D), lambda i: (0, 0)),
            pl.BlockSpec((1, D), lambda i: (0, 0)),
            pl.BlockSpec((D, D), lambda i: (0, 0)),
            pl.BlockSpec((1, D), lambda i: (0, 0)),
            pl.BlockSpec((1, D), lambda i: (0, 0)),
        ],
        out_specs=pl.BlockSpec((BLK, D), lambda i: (i, 0)),
        out_shape=jax.ShapeDtypeStruct((N, D), jnp.float32),
    )(cnt_part, part, xin, wl, bl, wr, gam, bet)


def kernel(x, edge_index, W1l, b1l, W1r, bn_gamma, bn_beta, W2l, b2l, W2r):
    src = edge_index[0].astype(jnp.int32)
    dst = edge_index[1].astype(jnp.int32)

    part1, cntp = _agg_l1(x, src, dst)
    h = _dense(cntp, part1, x, W1l, b1l.reshape(1, D), W1r,
               bn_gamma.reshape(1, D), bn_beta.reshape(1, D), True)
    part2 = _agg_l2(h, src, dst)
    out = _dense(cntp, part2, h, W2l, b2l.reshape(1, D), W2r,
                 bn_gamma.reshape(1, D), bn_beta.reshape(1, D), False)
    return out


# speedup vs baseline: 1.0806x; 1.0109x over previous
import functools

import jax
import jax.numpy as jnp
from jax import lax
from jax.experimental import pallas as pl
from jax.experimental.pallas import tpu as pltpu
from jax.experimental.pallas import tpu_sc as plsc

N = 10000
D = 128
DP = 144
E = 320000
NC, NS = 2, 16
NW = NC * NS
EPW = E // NW
CHUNK = 80
NCHUNK = EPW // CHUNK
CW = 16
ZP = 10240
RPS = ZP // NS


def _make_agg(chunk, nbuf, count):
    mesh = plsc.VectorSubcoreMesh(core_axis_name="c", subcore_axis_name="s")
    nchunk = EPW // chunk
    out_type = jax.ShapeDtypeStruct((NC, ZP, D), jnp.float32)
    scratch = [
        pltpu.VMEM((EPW,), jnp.int32),
        pltpu.VMEM((EPW,), jnp.int32),
        [pltpu.VMEM((chunk, D), jnp.float32)] * nbuf,
        pltpu.VMEM_SHARED((ZP, D), jnp.float32),
        [pltpu.SemaphoreType.DMA] * nbuf,
    ]
    if count:
        out_type = (out_type, jax.ShapeDtypeStruct((NC, ZP, CW), jnp.float32))
        scratch += [
            pltpu.VMEM((chunk, CW), jnp.float32),
            pltpu.VMEM_SHARED((ZP, CW), jnp.float32),
        ]

    @functools.partial(pl.kernel, out_type=out_type, mesh=mesh,
                       compiler_params=pltpu.CompilerParams(
                           use_tc_tiling_on_sc=False),
                       scratch_types=scratch)
    def agg(table_hbm, src_hbm, dst_hbm, *rest):
        if count:
            (outf_hbm, outc_hbm, sidx_v, didx_v, rows, acc_sh, gsem,
             ones_v, cnt_sh) = rest
        else:
            outf_hbm, sidx_v, didx_v, rows, acc_sh, gsem = rest
        rows_a = rows[0]
        cid = lax.axis_index("c")
        sid = lax.axis_index("s")
        wid = sid * NC + cid

        @pl.loop(0, chunk)
        def _(r):
            @pl.loop(0, D, step=16)
            def _(j):
                rows_a[r, pl.ds(j, 16)] = jnp.zeros((16,), jnp.float32)

        base = pl.multiple_of(sid * RPS, 8)

        @pl.loop(0, RPS // chunk)
        def _(k):
            pltpu.sync_copy(
                rows_a, acc_sh.at[pl.ds(pl.multiple_of(base + k * chunk, 8), chunk)])

        if count:
            e0 = jnp.where(lax.iota(jnp.int32, 16) == 0,
                           jnp.float32(1.0), jnp.float32(0.0))

            @pl.loop(0, chunk)
            def _(r):
                ones_v[r, pl.ds(0, 16)] = e0

            @pl.loop(0, RPS // chunk)
            def _(k):
                pltpu.sync_copy(
                    rows_a.at[pl.ds(0, chunk), pl.ds(0, CW)],
                    cnt_sh.at[pl.ds(pl.multiple_of(base + k * chunk, 8), chunk)])

        plsc.subcore_barrier()

        pltpu.sync_copy(src_hbm.at[pl.ds(wid * EPW, EPW)], sidx_v)
        pltpu.sync_copy(dst_hbm.at[pl.ds(wid * EPW, EPW)], didx_v)

        @pl.loop(0, nchunk // nbuf)
        def _(p):
            i0 = nbuf * p
            gs = [pltpu.async_copy(
                      table_hbm.at[sidx_v.at[pl.ds((i0 + k) * chunk, chunk)]],
                      rows[k], gsem[k]) for k in range(nbuf)]
            for k in range(nbuf):
                gs[k].wait()
                didx = didx_v.at[pl.ds((i0 + k) * chunk, chunk)]
                pltpu.sync_copy(rows[k], acc_sh.at[didx], add=True)
                if count:
                    pltpu.sync_copy(ones_v, cnt_sh.at[didx], add=True)

        for i in range((nchunk // nbuf) * nbuf, nchunk):
            didx = didx_v.at[pl.ds(i * chunk, chunk)]
            pltpu.sync_copy(table_hbm.at[sidx_v.at[pl.ds(i * chunk, chunk)]],
                            rows_a)
            pltpu.sync_copy(rows_a, acc_sh.at[didx], add=True)
            if count:
                pltpu.sync_copy(ones_v, cnt_sh.at[didx], add=True)

        plsc.subcore_barrier()
        pltpu.sync_copy(acc_sh.at[pl.ds(base, RPS)],
                        outf_hbm.at[cid].at[pl.ds(base, RPS)])
        if count:
            pltpu.sync_copy(cnt_sh.at[pl.ds(base, RPS)],
                            outc_hbm.at[cid].at[pl.ds(base, RPS)])

    return agg


C1, C2 = 40, 80
_agg_l1 = _make_agg(C1, 3, True)
_agg_l2 = _make_agg(C2, 2, False)


def _dense_body(transition):
    def body(cnt_ref, part_ref, xin_ref, wl_ref, bl_ref, wr_ref, gam_ref,
             bet_ref, out_ref):
        s = part_ref[0] + part_ref[1]
        cnt = cnt_ref[0, :, 0:1] + cnt_ref[1, :, 0:1]
        mean = s / jnp.maximum(cnt, 1.0)
        dn = (((1,), (1,)), ((), ()))
        out = (lax.dot_general(mean, wl_ref[...], dn,
                               preferred_element_type=jnp.float32)
               + bl_ref[...]
               + lax.dot_general(xin_ref[...], wr_ref[...], dn,
                                 preferred_element_type=jnp.float32))
        nrm = jnp.sqrt(jnp.sum(out * out, axis=1, keepdims=True))
        out = out / jnp.maximum(nrm, 1e-12)
        if transition:
            scale = gam_ref[...] * (1.0 / jnp.sqrt(jnp.float32(1.0 + 1e-5)))
            out = jnp.maximum(out, 0.0) * scale + bet_ref[...]
        out_ref[...] = out

    return body


BLK = 2000


def _dense(cnt_part, part, xin, wl, bl, wr, gam, bet, transition):
    grid = (N // BLK,)
    return pl.pallas_call(
        _dense_body(transition),
        grid=grid,
        in_specs=[
            pl.BlockSpec((NC, BLK, CW), lambda i: (0, i, 0)),
            pl.BlockSpec((NC, BLK, D), lambda i: (0, i, 0)),
            pl.BlockSpec((BLK, D), lambda i: (i, 0)),
            pl.BlockSpec((D, D), lambda i: (0, 0)),
            pl.BlockSpec((1, D), lambda i: (0, 0)),
            pl.BlockSpec((D, D), lambda i: (0, 0)),
            pl.BlockSpec((1, D), lambda i: (0, 0)),
            pl.BlockSpec((1, D), lambda i: (0, 0)),
        ],
        out_specs=pl.BlockSpec((BLK, D), lambda i: (i, 0)),
        out_shape=jax.ShapeDtypeStruct((N, D), jnp.float32),
    )(cnt_part, part, xin, wl, bl, wr, gam, bet)


def kernel(x, edge_index, W1l, b1l, W1r, bn_gamma, bn_beta, W2l, b2l, W2r):
    src = edge_index[0].astype(jnp.int32)
    dst = edge_index[1].astype(jnp.int32)

    part1, cntp = _agg_l1(x, src, dst)
    h = _dense(cntp, part1, x, W1l, b1l.reshape(1, D), W1r,
               bn_gamma.reshape(1, D), bn_beta.reshape(1, D), True)
    part2 = _agg_l2(h, src, dst)
    out = _dense(cntp, part2, h, W2l, b2l.reshape(1, D), W2r,
                 bn_gamma.reshape(1, D), bn_beta.reshape(1, D), False)
    return out
